# baseline (speedup 1.0000x reference)
import functools

import jax
import jax.numpy as jnp
from jax import lax
from jax.experimental import pallas as pl
from jax.experimental.pallas import tpu as pltpu
from jax.experimental.pallas import tpu_sc as plsc

NUM_SC_CORES = 2
NUM_SC_SUBCORES = 16
NUM_WORKERS = NUM_SC_CORES * NUM_SC_SUBCORES


def _tables_body(bt_ref, pi_ref, pt_ref, ll_ref):
    bt = bt_ref[...]
    pi = pi_ref[...]
    m = bt.shape[0]
    sm_b = jax.nn.softmax(bt, axis=0)
    sm_pi = jax.nn.softmax(pi, axis=0)
    unnorm = sm_pi[None, :, :] * sm_b
    denom = jnp.sum(unnorm, axis=1)
    post = (unnorm / denom[:, None, :]).reshape(m, -1)
    d = post.shape[1]
    left = jnp.broadcast_to(post[:, None, :], (m, m, d)).reshape(m * m, d)
    right = jnp.broadcast_to(post[None, :, :], (m, m, d)).reshape(m * m, d)
    a16 = lax.bitcast_convert_type(left.astype(jnp.bfloat16), jnp.uint16)
    b16 = lax.bitcast_convert_type(right.astype(jnp.bfloat16), jnp.uint16)
    word = (b16.astype(jnp.uint32) << 16) | a16.astype(jnp.uint32)
    pt_ref[...] = jnp.zeros_like(pt_ref)
    pt_ref[:, :d] = word.astype(jnp.int32)
    ll_ref[...] = jnp.log(denom)


def _transpose_body(g_ref, out_ref, d: int):
    xi = g_ref[...]
    lo = lax.bitcast_convert_type(xi << 16, jnp.float32)
    hi = lax.bitcast_convert_type(xi & jnp.int32(-65536), jnp.float32)
    t_a = lo[:, :d].T
    t_b = hi[:, :d].T
    out_ref[...] = jnp.stack([t_a, t_b], axis=1).reshape(2 * d, t_a.shape[1])


def _ll_body(x_ref, tbl_ref, out_ref, m: int):
    xv = x_ref[0, :]
    tbl = tbl_ref[...]
    acc = jnp.zeros(out_ref.shape, jnp.float32)
    for mm in range(m):
        sel = (xv == mm)[None, :]
        acc = jnp.where(sel, tbl[mm][:, None], acc)
    out_ref[...] = acc


def _sc_gather(table_pad, idx, n, dpad, chunk):
    num_chunks = n // chunk
    iters = pl.cdiv(num_chunks, NUM_WORKERS)
    mesh = plsc.VectorSubcoreMesh(core_axis_name="c", subcore_axis_name="s")

    @functools.partial(
        pl.kernel,
        out_type=jax.ShapeDtypeStruct((n, dpad), jnp.int32),
        mesh=mesh,
        scratch_types=[
            pltpu.VMEM((chunk,), jnp.int32),
            pltpu.VMEM((chunk,), jnp.int32),
            pltpu.VMEM((chunk, dpad), jnp.int32),
            pltpu.VMEM((chunk, dpad), jnp.int32),
            pltpu.SemaphoreType.DMA,
            pltpu.SemaphoreType.DMA,
            pltpu.SemaphoreType.DMA,
            pltpu.SemaphoreType.DMA,
            pltpu.SemaphoreType.DMA,
            pltpu.SemaphoreType.DMA,
        ],
    )
    def gather_kernel(table_hbm, idx_hbm, out_hbm, idx0, idx1, rows0, rows1,
                      isem0, isem1, gsem0, gsem1, ssem0, ssem1):
        wid = lax.axis_index("s") * NUM_SC_CORES + lax.axis_index("c")
        rows = (rows0, rows1)
        idxb = (idx0, idx1)
        isem = (isem0, isem1)
        gsem = (gsem0, gsem1)
        ssem = (ssem0, ssem1)

        def chunk_of(k):
            return k * NUM_WORKERS + wid

        def idx_start(k):
            b = k % 2
            c = chunk_of(k)

            @pl.when(c < num_chunks)
            def _():
                pltpu.async_copy(idx_hbm.at[pl.ds(c * chunk, chunk)],
                                 idxb[b], isem[b])

        def idx_wait(k):
            b = k % 2
            c = chunk_of(k)

            @pl.when(c < num_chunks)
            def _():
                pltpu.make_async_copy(idx_hbm.at[pl.ds(c * chunk, chunk)],
                                      idxb[b], isem[b]).wait()

        def gather_start(k):
            b = k % 2
            c = chunk_of(k)

            @pl.when(c < num_chunks)
            def _():
                pltpu.async_copy(table_hbm.at[idxb[b]], rows[b], gsem[b])

        def gather_wait_store_start(k):
            b = k % 2
            c = chunk_of(k)

            @pl.when(c < num_chunks)
            def _():
                pltpu.make_async_copy(table_hbm.at[idxb[b]], rows[b],
                                      gsem[b]).wait()
                pltpu.async_copy(rows[b], out_hbm.at[pl.ds(c * chunk, chunk)],
                                 ssem[b])

        def store_drain(k):
            if k < 0:
                return
            b = k % 2
            c = chunk_of(k)

            @pl.when(c < num_chunks)
            def _():
                pltpu.make_async_copy(rows[b],
                                      out_hbm.at[pl.ds(c * chunk, chunk)],
                                      ssem[b]).wait()

        idx_start(0)
        for k in range(iters):
            store_drain(k - 2)
            idx_wait(k)
            gather_start(k)
            idx_start(k + 1)
            gather_wait_store_start(k)
        store_drain(iters - 2)
        store_drain(iters - 1)

    return gather_kernel(table_pad, idx)


def kernel(x, edge_index, h_prev, B, Pi):
    c, m, ngen = B.shape
    n = x.shape[0]
    d = c * ngen
    dpad = 256

    x = x.astype(jnp.int32)
    bt = jnp.transpose(B, (1, 0, 2))

    pair_tbl, ll_tbl = pl.pallas_call(
        _tables_body,
        out_shape=(
            jax.ShapeDtypeStruct((m * m, dpad), jnp.int32),
            jax.ShapeDtypeStruct((m, ngen), jnp.float32),
        ),
    )(bt, Pi)

    n_out = ((n + 127) // 128) * 128
    half = n_out // 2
    xpad = jnp.pad(x, (0, n_out - n))
    pair_idx = xpad[:half] * m + xpad[half:]

    chunk = 128
    assert half % chunk == 0 and chunk % 8 == 0
    gp = _sc_gather(pair_tbl, pair_idx, half, dpad, chunk)

    bnp = 2176
    gi = half // bnp
    p2d = pl.pallas_call(
        functools.partial(_transpose_body, d=d),
        grid=(gi,),
        in_specs=[pl.BlockSpec((bnp, dpad), lambda i: (i, 0))],
        out_specs=pl.BlockSpec((2 * d, bnp), lambda i: (0, i)),
        out_shape=jax.ShapeDtypeStruct((2 * d, half), jnp.float32),
    )(gp)
    p2d = p2d.reshape(d, n_out)

    bn = 2048
    grid = pl.cdiv(n, bn)
    x2 = x.reshape(1, n)
    l2d = pl.pallas_call(
        functools.partial(_ll_body, m=m),
        grid=(grid,),
        in_specs=[
            pl.BlockSpec((1, bn), lambda i: (0, i)),
            pl.BlockSpec((m, ngen), lambda i: (0, 0)),
        ],
        out_specs=pl.BlockSpec((ngen, bn), lambda i: (0, i)),
        out_shape=jax.ShapeDtypeStruct((ngen, n), jnp.float32),
    )(x2, ll_tbl)

    log_likelihood = jnp.transpose(l2d, (1, 0))[:, None, :]
    posterior = jnp.transpose(p2d.reshape(c, ngen, -1), (2, 0, 1))[:n]
    return (log_likelihood, posterior)


# speedup vs baseline: 1.0513x; 1.0513x over previous
import functools

import jax
import jax.numpy as jnp
from jax import lax
from jax.experimental import pallas as pl
from jax.experimental.pallas import tpu as pltpu
from jax.experimental.pallas import tpu_sc as plsc

NUM_SC_CORES = 2
NUM_SC_SUBCORES = 16
NUM_WORKERS = NUM_SC_CORES * NUM_SC_SUBCORES


def _tables_body(bt_ref, pi_ref, pt_ref, ll_ref):
    bt = bt_ref[...]
    pi = pi_ref[...]
    m = bt.shape[0]
    sm_b = jax.nn.softmax(bt, axis=0)
    sm_pi = jax.nn.softmax(pi, axis=0)
    unnorm = sm_pi[None, :, :] * sm_b
    denom = jnp.sum(unnorm, axis=1)
    post = (unnorm / denom[:, None, :]).reshape(m, -1)
    d = post.shape[1]
    left = jnp.broadcast_to(post[:, None, :], (m, m, d)).reshape(m * m, d)
    right = jnp.broadcast_to(post[None, :, :], (m, m, d)).reshape(m * m, d)
    pt_ref[...] = jnp.zeros_like(pt_ref)
    pt_ref[:, :d] = left
    pt_ref[:, 256:256 + d] = right
    ll_ref[...] = jnp.log(denom)


def _transpose_body(g_ref, out_ref, d: int):
    out_ref[...] = g_ref[...][:, :d].T


def _ll_body(x_ref, tbl_ref, out_ref, m: int):
    xv = x_ref[0, :]
    tbl = tbl_ref[...]
    acc = jnp.zeros(out_ref.shape, jnp.float32)
    for mm in range(m):
        sel = (xv == mm)[None, :]
        acc = jnp.where(sel, tbl[mm][:, None], acc)
    out_ref[...] = acc


def _sc_gather(table_pad, idx, n, dpad, chunk):
    num_chunks = n // chunk
    iters = pl.cdiv(num_chunks, NUM_WORKERS)
    mesh = plsc.VectorSubcoreMesh(core_axis_name="c", subcore_axis_name="s")

    @functools.partial(
        pl.kernel,
        out_type=jax.ShapeDtypeStruct((n, dpad), jnp.float32),
        mesh=mesh,
        scratch_types=[
            pltpu.VMEM((chunk,), jnp.int32),
            pltpu.VMEM((chunk,), jnp.int32),
            pltpu.VMEM((chunk, dpad), jnp.float32),
            pltpu.VMEM((chunk, dpad), jnp.float32),
            pltpu.SemaphoreType.DMA,
            pltpu.SemaphoreType.DMA,
            pltpu.SemaphoreType.DMA,
            pltpu.SemaphoreType.DMA,
            pltpu.SemaphoreType.DMA,
            pltpu.SemaphoreType.DMA,
        ],
    )
    def gather_kernel(table_hbm, idx_hbm, out_hbm, idx0, idx1, rows0, rows1,
                      isem0, isem1, gsem0, gsem1, ssem0, ssem1):
        wid = lax.axis_index("s") * NUM_SC_CORES + lax.axis_index("c")
        rows = (rows0, rows1)
        idxb = (idx0, idx1)
        isem = (isem0, isem1)
        gsem = (gsem0, gsem1)
        ssem = (ssem0, ssem1)

        def chunk_of(k):
            return k * NUM_WORKERS + wid

        def idx_start(k):
            b = k % 2
            c = chunk_of(k)

            @pl.when(c < num_chunks)
            def _():
                pltpu.async_copy(idx_hbm.at[pl.ds(c * chunk, chunk)],
                                 idxb[b], isem[b])

        def idx_wait(k):
            b = k % 2
            c = chunk_of(k)

            @pl.when(c < num_chunks)
            def _():
                pltpu.make_async_copy(idx_hbm.at[pl.ds(c * chunk, chunk)],
                                      idxb[b], isem[b]).wait()

        def gather_start(k):
            b = k % 2
            c = chunk_of(k)

            @pl.when(c < num_chunks)
            def _():
                pltpu.async_copy(table_hbm.at[idxb[b]], rows[b], gsem[b])

        def gather_wait_store_start(k):
            b = k % 2
            c = chunk_of(k)

            @pl.when(c < num_chunks)
            def _():
                pltpu.make_async_copy(table_hbm.at[idxb[b]], rows[b],
                                      gsem[b]).wait()
                pltpu.async_copy(rows[b], out_hbm.at[pl.ds(c * chunk, chunk)],
                                 ssem[b])

        def store_drain(k):
            if k < 0:
                return
            b = k % 2
            c = chunk_of(k)

            @pl.when(c < num_chunks)
            def _():
                pltpu.make_async_copy(rows[b],
                                      out_hbm.at[pl.ds(c * chunk, chunk)],
                                      ssem[b]).wait()

        idx_start(0)
        for k in range(iters):
            store_drain(k - 2)
            idx_wait(k)
            gather_start(k)
            idx_start(k + 1)
            gather_wait_store_start(k)
        store_drain(iters - 2)
        store_drain(iters - 1)

    return gather_kernel(table_pad, idx)


def kernel(x, edge_index, h_prev, B, Pi):
    c, m, ngen = B.shape
    n = x.shape[0]
    d = c * ngen
    dpad = 512

    x = x.astype(jnp.int32)
    bt = jnp.transpose(B, (1, 0, 2))

    pair_tbl, ll_tbl = pl.pallas_call(
        _tables_body,
        out_shape=(
            jax.ShapeDtypeStruct((m * m, dpad), jnp.float32),
            jax.ShapeDtypeStruct((m, ngen), jnp.float32),
        ),
    )(bt, Pi)

    n_out = ((n + 127) // 128) * 128
    half = n_out // 2
    xpad = jnp.pad(x, (0, n_out - n))
    pair_idx = xpad[:half] * m + xpad[half:]

    chunk = 64
    assert half % chunk == 0 and chunk % 8 == 0
    gp = _sc_gather(pair_tbl, pair_idx, half, dpad, chunk)

    bnp = 2176
    gi = half // bnp
    p2d = pl.pallas_call(
        functools.partial(_transpose_body, d=d),
        grid=(gi, 2),
        in_specs=[pl.BlockSpec((bnp, 256), lambda i, h: (i, h))],
        out_specs=pl.BlockSpec((d, bnp), lambda i, h: (0, h * gi + i)),
        out_shape=jax.ShapeDtypeStruct((d, n_out), jnp.float32),
    )(gp)

    bn = 2048
    grid = pl.cdiv(n, bn)
    x2 = x.reshape(1, n)
    l2d = pl.pallas_call(
        functools.partial(_ll_body, m=m),
        grid=(grid,),
        in_specs=[
            pl.BlockSpec((1, bn), lambda i: (0, i)),
            pl.BlockSpec((m, ngen), lambda i: (0, 0)),
        ],
        out_specs=pl.BlockSpec((ngen, bn), lambda i: (0, i)),
        out_shape=jax.ShapeDtypeStruct((ngen, n), jnp.float32),
    )(x2, ll_tbl)

    log_likelihood = jnp.transpose(l2d, (1, 0))[:, None, :]
    posterior = jnp.transpose(p2d.reshape(c, ngen, -1), (2, 0, 1))[:n]
    return (log_likelihood, posterior)
